# baseline (speedup 1.0000x reference)
import jax
import jax.numpy as jnp
from jax.experimental import pallas as pl
from jax.experimental.pallas import tpu as pltpu

_B, _T, _A, _D = 8, 256, 32, 256
_NZ = 256
_NL2 = 40
_HALF = _B // 2
_R = _HALF * _T

_PREC = jax.lax.Precision.DEFAULT


def _dot(a, b, precision=_PREC):
    return jax.lax.dot_general(
        a, b, (((1,), (0,)), ((), ())),
        precision=precision, preferred_element_type=jnp.float32)


def _fwd_kernel(actions_ref, enc_in_W_ref, enc_in_b_ref, W_ref, b_ref,
                ln_g_ref, ln_b_ref, cz_ref, czT_ref, out_W_ref, out_b_ref,
                out_ref, x_s, tmod_s):
    j = pl.program_id(1)

    @pl.when(j == 0)
    def _init():
        x_s[:] = _dot(actions_ref[0], enc_in_W_ref[:]) + enc_in_b_ref[:]
        tmod_s[:] = jax.lax.rem(
            jax.lax.broadcasted_iota(jnp.int32, (_R, _D), 0), jnp.int32(_T))

    @pl.when(j == 20)
    def _vq():
        x = x_s[:]
        mu = jnp.mean(x, axis=-1, keepdims=True)
        var = jnp.mean((x - mu) ** 2, axis=-1, keepdims=True)
        e = (x - mu) / jnp.sqrt(var + 1e-5) * ln_g_ref[:] + ln_b_ref[:]
        rs = jnp.sum(e * e, axis=-1, keepdims=True)
        cn = jnp.sum(czT_ref[:] * czT_ref[:], axis=0, keepdims=True)
        dz = rs - 2.0 * _dot(e, czT_ref[:]) + cn
        idx = jnp.argmin(dz, axis=-1, keepdims=True)
        lanes = jax.lax.broadcasted_iota(jnp.int32, (_R, _NZ), 1)
        onehot = (lanes == idx).astype(jnp.float32)
        x_s[:] = _dot(onehot, cz_ref[:], precision=jax.lax.Precision.HIGHEST)

    d = jax.lax.shift_left(jnp.int32(1), j % 10)
    x = x_s[:]

    def _full_tap():
        tmod = tmod_s[:]
        shm = jnp.where(tmod >= d, pltpu.roll(x, d, axis=0), 0.0)
        shp = jnp.where(tmod < _T - d, pltpu.roll(x, _R - d, axis=0), 0.0)
        patches = jnp.concatenate([shm, x, shp], axis=-1)
        return _dot(patches, W_ref[0])

    def _center_tap():
        return _dot(x, W_ref[0, _D:2 * _D, :])

    u = jax.lax.cond(d < _T, _full_tap, _center_tap)
    x_s[:] = x + jnp.maximum(u + b_ref[0], 0.0)

    @pl.when(j == _NL2 - 1)
    def _out():
        out_ref[0] = _dot(x_s[:], out_W_ref[:]) + out_b_ref[:]


def kernel(actions, enc_in_W, enc_in_b, enc_convW, enc_convb, ln_g, ln_b,
           codebook_z, codebook_q, dec_convW, dec_convb, out_W, out_b):
    del codebook_q
    f32 = jnp.float32
    Wcat = jnp.concatenate([enc_convW, dec_convW], axis=0)
    W_all = jnp.transpose(Wcat, (0, 3, 2, 1)).reshape(_NL2, 3 * _D, _D)
    b_all = jnp.concatenate([enc_convb, dec_convb], axis=0).reshape(_NL2, 1, _D)
    actions3 = actions.reshape(2, _R, _A)

    grid = (2, _NL2)
    recon = pl.pallas_call(
        _fwd_kernel,
        grid=grid,
        in_specs=[
            pl.BlockSpec((1, _R, _A), lambda i, j: (i, 0, 0)),
            pl.BlockSpec((_A, _D), lambda i, j: (0, 0)),
            pl.BlockSpec((1, _D), lambda i, j: (0, 0)),
            pl.BlockSpec((1, 3 * _D, _D), lambda i, j: (j, 0, 0)),
            pl.BlockSpec((1, 1, _D), lambda i, j: (j, 0, 0)),
            pl.BlockSpec((1, _D), lambda i, j: (0, 0)),
            pl.BlockSpec((1, _D), lambda i, j: (0, 0)),
            pl.BlockSpec((_NZ, _D), lambda i, j: (0, 0)),
            pl.BlockSpec((_D, _NZ), lambda i, j: (0, 0)),
            pl.BlockSpec((_D, _A), lambda i, j: (0, 0)),
            pl.BlockSpec((1, _A), lambda i, j: (0, 0)),
        ],
        out_specs=pl.BlockSpec((1, _R, _A), lambda i, j: (i, 0, 0)),
        out_shape=jax.ShapeDtypeStruct((2, _R, _A), f32),
        scratch_shapes=[
            pltpu.VMEM((_R, _D), f32),
            pltpu.VMEM((_R, _D), jnp.int32),
        ],
        compiler_params=pltpu.CompilerParams(
            dimension_semantics=("parallel", "arbitrary")),
    )(actions3, enc_in_W.astype(f32), enc_in_b.reshape(1, _D),
      W_all, b_all, ln_g.reshape(1, _D), ln_b.reshape(1, _D),
      codebook_z, codebook_z.T, out_W, out_b.reshape(1, _A))
    return recon.reshape(_B, _T, _A)


# speedup vs baseline: 1.0683x; 1.0683x over previous
import jax
import jax.numpy as jnp
from jax.experimental import pallas as pl
from jax.experimental.pallas import tpu as pltpu

_B, _T, _A, _D = 8, 256, 32, 256
_NZ = 256
_NL2 = 40
_HALF = _B // 2
_R = _HALF * _T

_PREC = jax.lax.Precision.DEFAULT


def _dot(a, b, precision=_PREC):
    return jax.lax.dot_general(
        a, b, (((1,), (0,)), ((), ())),
        precision=precision, preferred_element_type=jnp.float32)


def _fwd_kernel(actions_ref, enc_in_W_ref, enc_in_b_ref, W_ref, b_ref,
                ln_g_ref, ln_b_ref, cz_ref, czT_ref, out_W_ref, out_b_ref,
                out_ref, x_s, tmod_s):
    j = pl.program_id(1)

    @pl.when(j == 0)
    def _init():
        x_s[:] = _dot(actions_ref[0], enc_in_W_ref[:]) + enc_in_b_ref[:]
        tmod_s[:] = jax.lax.rem(
            jax.lax.broadcasted_iota(jnp.int32, (_R, _D), 0), jnp.int32(_T))

    @pl.when(j == 20)
    def _vq():
        x = x_s[:]
        mu = jnp.mean(x, axis=-1, keepdims=True)
        var = jnp.mean((x - mu) ** 2, axis=-1, keepdims=True)
        e = (x - mu) / jnp.sqrt(var + 1e-5) * ln_g_ref[:] + ln_b_ref[:]
        rs = jnp.sum(e * e, axis=-1, keepdims=True)
        cn = jnp.sum(czT_ref[:] * czT_ref[:], axis=0, keepdims=True)
        dz = rs - 2.0 * _dot(e, czT_ref[:]) + cn
        idx = jnp.argmin(dz, axis=-1, keepdims=True)
        lanes = jax.lax.broadcasted_iota(jnp.int32, (_R, _NZ), 1)
        onehot = (lanes == idx).astype(jnp.float32)
        x_s[:] = _dot(onehot, cz_ref[:], precision=jax.lax.Precision.HIGHEST)

    d = jax.lax.shift_left(jnp.int32(1), j % 10)
    x = x_s[:]

    tmod = tmod_s[:]
    shm = jnp.where(tmod >= d, pltpu.roll(x, d, axis=0), 0.0)
    shp = jnp.where(tmod < _T - d, pltpu.roll(x, _R - d, axis=0), 0.0)
    patches = jnp.concatenate([shm, x, shp], axis=-1)
    u = _dot(patches, W_ref[0])
    x_s[:] = x + jnp.maximum(u + b_ref[0], 0.0)

    @pl.when(j == _NL2 - 1)
    def _out():
        out_ref[0] = _dot(x_s[:], out_W_ref[:]) + out_b_ref[:]


def kernel(actions, enc_in_W, enc_in_b, enc_convW, enc_convb, ln_g, ln_b,
           codebook_z, codebook_q, dec_convW, dec_convb, out_W, out_b):
    del codebook_q
    f32 = jnp.float32
    Wcat = jnp.concatenate([enc_convW, dec_convW], axis=0)
    W_all = jnp.transpose(Wcat, (0, 3, 2, 1)).reshape(_NL2, 3 * _D, _D)
    b_all = jnp.concatenate([enc_convb, dec_convb], axis=0).reshape(_NL2, 1, _D)
    actions3 = actions.reshape(2, _R, _A)

    grid = (2, _NL2)
    recon = pl.pallas_call(
        _fwd_kernel,
        grid=grid,
        in_specs=[
            pl.BlockSpec((1, _R, _A), lambda i, j: (i, 0, 0)),
            pl.BlockSpec((_A, _D), lambda i, j: (0, 0)),
            pl.BlockSpec((1, _D), lambda i, j: (0, 0)),
            pl.BlockSpec((1, 3 * _D, _D), lambda i, j: (j, 0, 0)),
            pl.BlockSpec((1, 1, _D), lambda i, j: (j, 0, 0)),
            pl.BlockSpec((1, _D), lambda i, j: (0, 0)),
            pl.BlockSpec((1, _D), lambda i, j: (0, 0)),
            pl.BlockSpec((_NZ, _D), lambda i, j: (0, 0)),
            pl.BlockSpec((_D, _NZ), lambda i, j: (0, 0)),
            pl.BlockSpec((_D, _A), lambda i, j: (0, 0)),
            pl.BlockSpec((1, _A), lambda i, j: (0, 0)),
        ],
        out_specs=pl.BlockSpec((1, _R, _A), lambda i, j: (i, 0, 0)),
        out_shape=jax.ShapeDtypeStruct((2, _R, _A), f32),
        scratch_shapes=[
            pltpu.VMEM((_R, _D), f32),
            pltpu.VMEM((_R, _D), jnp.int32),
        ],
        compiler_params=pltpu.CompilerParams(
            dimension_semantics=("parallel", "arbitrary")),
    )(actions3, enc_in_W.astype(f32), enc_in_b.reshape(1, _D),
      W_all, b_all, ln_g.reshape(1, _D), ln_b.reshape(1, _D),
      codebook_z, codebook_z.T, out_W, out_b.reshape(1, _A))
    return recon.reshape(_B, _T, _A)


# speedup vs baseline: 1.5514x; 1.4522x over previous
import jax
import jax.numpy as jnp
from jax.experimental import pallas as pl
from jax.experimental.pallas import tpu as pltpu

_B, _T, _A, _D = 8, 256, 32, 256
_NZ = 256
_NL2 = 40
_LPS = 10
_HALF = _B // 2
_R = _HALF * _T

_PREC = jax.lax.Precision.DEFAULT


def _dot(a, b, precision=_PREC):
    return jax.lax.dot_general(
        a, b, (((1,), (0,)), ((), ())),
        precision=precision, preferred_element_type=jnp.float32)


def _fwd_kernel(actions_ref, enc_in_W_ref, enc_in_b_ref, W_ref, b_ref,
                ln_g_ref, ln_b_ref, cz_ref, czT_ref, out_W_ref, out_b_ref,
                out_ref, x_s, tmod_s):
    j = pl.program_id(1)

    @pl.when(j == 0)
    def _init():
        x_s[:] = _dot(actions_ref[0], enc_in_W_ref[:]) + enc_in_b_ref[:]
        tmod_s[:] = jax.lax.rem(
            jax.lax.broadcasted_iota(jnp.int32, (_R, _D), 0), jnp.int32(_T))

    @pl.when(j == 2)
    def _vq():
        x = x_s[:]
        mu = jnp.mean(x, axis=-1, keepdims=True)
        var = jnp.mean((x - mu) ** 2, axis=-1, keepdims=True)
        e = (x - mu) / jnp.sqrt(var + 1e-5) * ln_g_ref[:] + ln_b_ref[:]
        rs = jnp.sum(e * e, axis=-1, keepdims=True)
        cn = jnp.sum(czT_ref[:] * czT_ref[:], axis=0, keepdims=True)
        dz = rs - 2.0 * _dot(e, czT_ref[:]) + cn
        idx = jnp.argmin(dz, axis=-1, keepdims=True)
        lanes = jax.lax.broadcasted_iota(jnp.int32, (_R, _NZ), 1)
        onehot = (lanes == idx).astype(jnp.float32)
        x_s[:] = _dot(onehot, cz_ref[:], precision=jax.lax.Precision.HIGHEST)

    x = x_s[:]
    tm = tmod_s[:]
    for k in range(_LPS):
        d = 1 << k
        bk = b_ref[0, k:k + 1, :]
        if d < _T:
            shm = jnp.where(tm >= d, pltpu.roll(x, d, axis=0), 0.0)
            shp = jnp.where(tm < _T - d, pltpu.roll(x, _R - d, axis=0), 0.0)
            patches = jnp.concatenate([shm, x, shp], axis=-1)
            u = _dot(patches, W_ref[k])
        else:
            u = _dot(x, W_ref[k, _D:2 * _D, :])
        x = x + jnp.maximum(u + bk, 0.0)
    x_s[:] = x

    @pl.when(j == _NL2 // _LPS - 1)
    def _out():
        out_ref[0] = _dot(x, out_W_ref[:]) + out_b_ref[:]


def kernel(actions, enc_in_W, enc_in_b, enc_convW, enc_convb, ln_g, ln_b,
           codebook_z, codebook_q, dec_convW, dec_convb, out_W, out_b):
    del codebook_q
    f32 = jnp.float32
    Wcat = jnp.concatenate([enc_convW, dec_convW], axis=0)
    W_all = jnp.transpose(Wcat, (0, 3, 2, 1)).reshape(_NL2, 3 * _D, _D)
    b_all = jnp.concatenate([enc_convb, dec_convb], axis=0).reshape(
        _NL2 // _LPS, _LPS, _D)
    actions3 = actions.reshape(2, _R, _A)

    grid = (2, _NL2 // _LPS)
    recon = pl.pallas_call(
        _fwd_kernel,
        grid=grid,
        in_specs=[
            pl.BlockSpec((1, _R, _A), lambda i, j: (i, 0, 0)),
            pl.BlockSpec((_A, _D), lambda i, j: (0, 0)),
            pl.BlockSpec((1, _D), lambda i, j: (0, 0)),
            pl.BlockSpec((_LPS, 3 * _D, _D), lambda i, j: (j, 0, 0)),
            pl.BlockSpec((1, _LPS, _D), lambda i, j: (j, 0, 0)),
            pl.BlockSpec((1, _D), lambda i, j: (0, 0)),
            pl.BlockSpec((1, _D), lambda i, j: (0, 0)),
            pl.BlockSpec((_NZ, _D), lambda i, j: (0, 0)),
            pl.BlockSpec((_D, _NZ), lambda i, j: (0, 0)),
            pl.BlockSpec((_D, _A), lambda i, j: (0, 0)),
            pl.BlockSpec((1, _A), lambda i, j: (0, 0)),
        ],
        out_specs=pl.BlockSpec((1, _R, _A), lambda i, j: (i, 0, 0)),
        out_shape=jax.ShapeDtypeStruct((2, _R, _A), f32),
        scratch_shapes=[
            pltpu.VMEM((_R, _D), f32),
            pltpu.VMEM((_R, _D), jnp.int32),
        ],
        compiler_params=pltpu.CompilerParams(
            dimension_semantics=("parallel", "arbitrary")),
    )(actions3, enc_in_W.astype(f32), enc_in_b.reshape(1, _D),
      W_all, b_all, ln_g.reshape(1, _D), ln_b.reshape(1, _D),
      codebook_z, codebook_z.T, out_W, out_b.reshape(1, _A))
    return recon.reshape(_B, _T, _A)


# speedup vs baseline: 1.8874x; 1.2166x over previous
import jax
import jax.numpy as jnp
from jax.experimental import pallas as pl
from jax.experimental.pallas import tpu as pltpu

_B, _T, _A, _D = 8, 256, 32, 256
_NZ = 256
_NL2 = 40
_LPS = 10
_HALF = _B // 2
_R = _HALF * _T

_PREC = jax.lax.Precision.DEFAULT


def _dot(a, b, precision=_PREC):
    return jax.lax.dot_general(
        a, b, (((1,), (0,)), ((), ())),
        precision=precision, preferred_element_type=jnp.float32)


def _fwd_kernel(actions_ref, enc_in_W_ref, enc_in_b_ref, W_ref, b_ref,
                ln_g_ref, ln_b_ref, cz_ref, czT_ref, out_W_ref, out_b_ref,
                out_ref, x_s, tmod_s):
    j = pl.program_id(1)

    @pl.when(j == 0)
    def _init():
        x_s[:] = _dot(actions_ref[0], enc_in_W_ref[:]) + enc_in_b_ref[:]
        tmod_s[:] = jax.lax.rem(
            jax.lax.broadcasted_iota(jnp.int32, (_R, _D), 0), jnp.int32(_T))

    @pl.when(j == 2)
    def _vq():
        x = x_s[:]
        mu = jnp.mean(x, axis=-1, keepdims=True)
        var = jnp.mean((x - mu) ** 2, axis=-1, keepdims=True)
        e = (x - mu) / jnp.sqrt(var + 1e-5) * ln_g_ref[:] + ln_b_ref[:]
        rs = jnp.sum(e * e, axis=-1, keepdims=True)
        cn = jnp.sum(czT_ref[:] * czT_ref[:], axis=0, keepdims=True)
        dz = rs - 2.0 * _dot(e, czT_ref[:]) + cn
        idx = jnp.argmin(dz, axis=-1, keepdims=True)
        lanes = jax.lax.broadcasted_iota(jnp.int32, (_R, _NZ), 1)
        onehot = (lanes == idx).astype(jnp.float32)
        x_s[:] = _dot(onehot, cz_ref[:], precision=jax.lax.Precision.HIGHEST)

    x = x_s[:]
    tm = tmod_s[:]
    for k in range(_LPS):
        d = 1 << k
        bk = b_ref[0, k:k + 1, :]
        if d < _T:
            a = _dot(x, W_ref[k, 0:_D, :])
            a = jnp.where(tm >= d, pltpu.roll(a, d, axis=0), 0.0)
            a = a + _dot(x, W_ref[k, _D:2 * _D, :])
            c = _dot(x, W_ref[k, 2 * _D:3 * _D, :])
            c = jnp.where(tm < _T - d, pltpu.roll(c, _R - d, axis=0), 0.0)
            u = a + c
        else:
            u = _dot(x, W_ref[k, _D:2 * _D, :])
        x = x + jnp.maximum(u + bk, 0.0)
    x_s[:] = x

    @pl.when(j == _NL2 // _LPS - 1)
    def _out():
        out_ref[0] = _dot(x, out_W_ref[:]) + out_b_ref[:]


def kernel(actions, enc_in_W, enc_in_b, enc_convW, enc_convb, ln_g, ln_b,
           codebook_z, codebook_q, dec_convW, dec_convb, out_W, out_b):
    del codebook_q
    f32 = jnp.float32
    Wcat = jnp.concatenate([enc_convW, dec_convW], axis=0)
    W_all = jnp.transpose(Wcat, (0, 3, 2, 1)).reshape(_NL2, 3 * _D, _D)
    b_all = jnp.concatenate([enc_convb, dec_convb], axis=0).reshape(
        _NL2 // _LPS, _LPS, _D)
    actions3 = actions.reshape(2, _R, _A)

    grid = (2, _NL2 // _LPS)
    recon = pl.pallas_call(
        _fwd_kernel,
        grid=grid,
        in_specs=[
            pl.BlockSpec((1, _R, _A), lambda i, j: (i, 0, 0)),
            pl.BlockSpec((_A, _D), lambda i, j: (0, 0)),
            pl.BlockSpec((1, _D), lambda i, j: (0, 0)),
            pl.BlockSpec((_LPS, 3 * _D, _D), lambda i, j: (j, 0, 0)),
            pl.BlockSpec((1, _LPS, _D), lambda i, j: (j, 0, 0)),
            pl.BlockSpec((1, _D), lambda i, j: (0, 0)),
            pl.BlockSpec((1, _D), lambda i, j: (0, 0)),
            pl.BlockSpec((_NZ, _D), lambda i, j: (0, 0)),
            pl.BlockSpec((_D, _NZ), lambda i, j: (0, 0)),
            pl.BlockSpec((_D, _A), lambda i, j: (0, 0)),
            pl.BlockSpec((1, _A), lambda i, j: (0, 0)),
        ],
        out_specs=pl.BlockSpec((1, _R, _A), lambda i, j: (i, 0, 0)),
        out_shape=jax.ShapeDtypeStruct((2, _R, _A), f32),
        scratch_shapes=[
            pltpu.VMEM((_R, _D), f32),
            pltpu.VMEM((_R, _D), jnp.int32),
        ],
        compiler_params=pltpu.CompilerParams(
            dimension_semantics=("parallel", "arbitrary")),
    )(actions3, enc_in_W.astype(f32), enc_in_b.reshape(1, _D),
      W_all, b_all, ln_g.reshape(1, _D), ln_b.reshape(1, _D),
      codebook_z, codebook_z.T, out_W, out_b.reshape(1, _A))
    return recon.reshape(_B, _T, _A)
